# Initial kernel scaffold; baseline (speedup 1.0000x reference)
import jax
import jax.numpy as jnp
from jax.experimental import pallas as pl


def kernel(x, llm_query, vit_query, static_keys_llm, static_keys_vit, rank_A_pool, rank_B_pool):
    raise NotImplementedError("write your pallas kernel here")


# speedup vs baseline: 1.1950x; 1.1950x over previous
import functools

import jax
import jax.numpy as jnp
from jax.experimental import pallas as pl

B, S, D_LLM, D_VIT, K, TOPK = 4, 2048, 2048, 1024, 512, 128
TS = 256


def _score_mask_kernel(lq_ref, vq_ref, kl_ref, kv_ref, mask_ref):
    k_ratio = float(D_VIT) / float(D_LLM)
    score = jax.lax.dot_general(
        lq_ref[...], kl_ref[...], (((1,), (1,)), ((), ())),
        preferred_element_type=jnp.float32)
    score = score + k_ratio * jax.lax.dot_general(
        vq_ref[...], kv_ref[...], (((1,), (1,)), ((), ())),
        preferred_element_type=jnp.float32)

    u = jax.lax.bitcast_convert_type(score, jnp.int32)
    key = u ^ ((u >> 31) & jnp.int32(0x7FFFFFFF))

    def count_ge(m):
        return jnp.sum((key >= m).astype(jnp.int32), axis=1, keepdims=True)

    int_min = jnp.full((B, 1), -2147483648, jnp.int32)
    zero = jnp.zeros((B, 1), jnp.int32)
    t = jnp.where(count_ge(zero) >= TOPK, zero, int_min)
    for b in range(30, -1, -1):
        cand = t | jnp.int32(1 << b)
        t = jnp.where(count_ge(cand) >= TOPK, cand, t)

    gt = key > t
    eq = key == t
    need = (TOPK - jnp.sum(gt.astype(jnp.int32), axis=1, keepdims=True)
            ).astype(jnp.float32)
    rows = jax.lax.broadcasted_iota(jnp.int32, (K, K), 0)
    cols = jax.lax.broadcasted_iota(jnp.int32, (K, K), 1)
    tri = (rows <= cols).astype(jnp.float32)
    csum = jnp.dot(eq.astype(jnp.float32), tri,
                   preferred_element_type=jnp.float32)
    mask = jnp.logical_or(gt, jnp.logical_and(eq, csum <= need))
    mask_ref[...] = mask.astype(jnp.float32)


def _lrp_matmul_kernel(x_ref, mask_ref, a_ref, b_ref, out_ref):
    xb = x_ref[0]
    t = jnp.dot(xb, a_ref[...], preferred_element_type=jnp.float32)
    t = t * mask_ref[0]
    out_ref[0] = xb + jnp.dot(t, b_ref[...],
                              preferred_element_type=jnp.float32)


@jax.jit
def kernel(x, llm_query, vit_query, static_keys_llm, static_keys_vit,
           rank_A_pool, rank_B_pool):
    mask = pl.pallas_call(
        _score_mask_kernel,
        out_shape=jax.ShapeDtypeStruct((B, K), jnp.float32),
    )(llm_query, vit_query, static_keys_llm, static_keys_vit)
    mask = mask.reshape(B, 1, K)

    out = pl.pallas_call(
        _lrp_matmul_kernel,
        grid=(B, S // TS),
        in_specs=[
            pl.BlockSpec((1, TS, D_LLM), lambda b, s: (b, s, 0)),
            pl.BlockSpec((1, 1, K), lambda b, s: (b, 0, 0)),
            pl.BlockSpec((D_LLM, K), lambda b, s: (0, 0)),
            pl.BlockSpec((K, D_LLM), lambda b, s: (0, 0)),
        ],
        out_specs=pl.BlockSpec((1, TS, D_LLM), lambda b, s: (b, s, 0)),
        out_shape=jax.ShapeDtypeStruct((B, S, D_LLM), jnp.float32),
    )(x, mask, rank_A_pool, rank_B_pool)
    return out
